# Initial kernel scaffold; baseline (speedup 1.0000x reference)
import jax
import jax.numpy as jnp
from jax.experimental import pallas as pl


def kernel(data, length, table, W, b):
    raise NotImplementedError("write your pallas kernel here")


# speedup vs baseline: 2.3547x; 2.3547x over previous
import functools

import jax
import jax.numpy as jnp
from jax import lax
from jax.experimental import pallas as pl
from jax.experimental.pallas import tpu as pltpu
from jax.experimental.pallas import tpu_sc as plsc

VOCAB = 1000000
EMB = 64
PAD = 16
BATCH = 16384
SEQ = 200

NC = 2
NS = 16
NW = NC * NS
ROWS_PER_W = BATCH // NW
CH = 8
NCHUNK = ROWS_PER_W // CH
TOK = CH * SEQ
GW = 100
G = TOK // GW

BV = 10000


def _proj_body(t_ref, w_ref, o_ref):
    o_ref[...] = jnp.dot(t_ref[...], w_ref[...],
                         preferred_element_type=jnp.float32)


def _project(table, wt):
    return pl.pallas_call(
        _proj_body,
        grid=(VOCAB // BV,),
        in_specs=[
            pl.BlockSpec((BV, EMB), lambda i: (i, 0)),
            pl.BlockSpec((EMB, PAD), lambda i: (0, 0)),
        ],
        out_specs=pl.BlockSpec((BV, PAD), lambda i: (i, 0)),
        out_shape=jax.ShapeDtypeStruct((VOCAB, PAD), jnp.float32),
    )(table, wt)


def _sc_pool(proj, data_r):
    mesh = plsc.VectorSubcoreMesh(core_axis_name="c", subcore_axis_name="s")

    @functools.partial(
        pl.kernel,
        mesh=mesh,
        compiler_params=pltpu.CompilerParams(use_tc_tiling_on_sc=False),
        out_type=jax.ShapeDtypeStruct((BATCH, PAD), jnp.float32),
        scratch_types=[
            pltpu.VMEM((G, GW), jnp.int32),
            pltpu.VMEM((TOK, PAD), jnp.float32),
            pltpu.VMEM((CH, PAD), jnp.float32),
            pltpu.SemaphoreType.DMA,
        ],
    )
    def k(proj_hbm, data_hbm, out_hbm, idx_v, rows_v, out_v, sem):
        cid = lax.axis_index("c")
        sid = lax.axis_index("s")
        wid = sid * NC + cid

        def chunk_body(c, carry):
            base = pl.multiple_of(wid * ROWS_PER_W + c * CH, CH)
            tbase = pl.multiple_of(base * SEQ // GW, CH * SEQ // GW)
            pltpu.sync_copy(data_hbm.at[pl.ds(tbase, G)], idx_v)
            for j in range(G):
                pltpu.async_copy(proj_hbm.at[idx_v.at[j]],
                                 rows_v.at[pl.ds(j * GW, GW)], sem)
            for j in range(G):
                pltpu.make_async_copy(proj_hbm.at[idx_v.at[j]],
                                      rows_v.at[pl.ds(j * GW, GW)], sem).wait()
            for r in range(CH):
                rb = r * SEQ

                def tok_body(t, accs):
                    a0, a1, a2, a3 = accs
                    p = rb + t * 4
                    return (a0 + rows_v[p, :], a1 + rows_v[p + 1, :],
                            a2 + rows_v[p + 2, :], a3 + rows_v[p + 3, :])

                z = jnp.zeros((PAD,), jnp.float32)
                a0, a1, a2, a3 = lax.fori_loop(0, SEQ // 4, tok_body,
                                               (z, z, z, z))
                out_v[r, :] = (a0 + a1) + (a2 + a3)
            pltpu.sync_copy(out_v, out_hbm.at[pl.ds(base, CH)])
            return carry

        lax.fori_loop(0, NCHUNK, chunk_body, 0)

    return k(proj, data_r)


def _fin_body(s_ref, l_ref, b_ref, o_ref):
    o_ref[...] = s_ref[...] / l_ref[...].astype(jnp.float32) + b_ref[...]


def _finalize(sums, length2d, bias16):
    return pl.pallas_call(
        _fin_body,
        out_shape=jax.ShapeDtypeStruct((BATCH, PAD), jnp.float32),
    )(sums, length2d, bias16)


def kernel(data, length, table, W, b):
    data = data.astype(jnp.int32)
    wt = jnp.zeros((EMB, PAD), jnp.float32).at[:, :2].set(W.T)
    bias16 = jnp.zeros((1, PAD), jnp.float32).at[0, :2].set(b)
    proj = _project(table, wt)
    data_r = data.reshape(BATCH * SEQ // GW, GW)
    sums = _sc_pool(proj, data_r)
    out = _finalize(sums, length.reshape(BATCH, 1), bias16)
    return out[:, :2]


# speedup vs baseline: 3.4034x; 1.4454x over previous
import functools

import jax
import jax.numpy as jnp
from jax import lax
from jax.experimental import pallas as pl
from jax.experimental.pallas import tpu as pltpu
from jax.experimental.pallas import tpu_sc as plsc

VOCAB = 1000000
EMB = 64
PAD = 16
BATCH = 16384
SEQ = 200

PACK = 8
VP = VOCAB // PACK
KP = EMB * PACK
NP = PAD * PACK

NC = 2
NS = 16
NW = NC * NS
ROWS_PER_W = BATCH // NW
CH = 8
NCHUNK = ROWS_PER_W // CH
TOK = CH * SEQ
GW = 100
G = TOK // GW

BV = 5000


def _proj_body(t_ref, m_ref, o_ref):
    o_ref[...] = jnp.dot(t_ref[...], m_ref[...],
                         preferred_element_type=jnp.float32)


def _project(table_p, m):
    return pl.pallas_call(
        _proj_body,
        grid=(VP // BV,),
        in_specs=[
            pl.BlockSpec((BV, KP), lambda i: (i, 0)),
            pl.BlockSpec((KP, NP), lambda i: (0, 0)),
        ],
        out_specs=pl.BlockSpec((BV, NP), lambda i: (i, 0)),
        out_shape=jax.ShapeDtypeStruct((VP, NP), jnp.float32),
    )(table_p, m)


def _sc_pool(proj, data_r):
    mesh = plsc.VectorSubcoreMesh(core_axis_name="c", subcore_axis_name="s")

    @functools.partial(
        pl.kernel,
        mesh=mesh,
        compiler_params=pltpu.CompilerParams(use_tc_tiling_on_sc=False),
        out_type=jax.ShapeDtypeStruct((BATCH, PAD), jnp.float32),
        scratch_types=[
            pltpu.VMEM((2, G, GW), jnp.int32),
            pltpu.VMEM((2, TOK, PAD), jnp.float32),
            pltpu.VMEM((CH, PAD), jnp.float32),
            pltpu.SemaphoreType.DMA,
            pltpu.SemaphoreType.DMA,
        ],
    )
    def k(proj_hbm, data_hbm, out_hbm, idx_v, rows_v, out_v, sem0, sem1):
        cid = lax.axis_index("c")
        sid = lax.axis_index("s")
        wid = sid * NC + cid
        sems = (sem0, sem1)

        def stage(c, p):
            base = pl.multiple_of(wid * ROWS_PER_W + c * CH, CH)
            tbase = pl.multiple_of(base * SEQ // GW, CH * SEQ // GW)
            pltpu.sync_copy(data_hbm.at[pl.ds(tbase, G)], idx_v.at[p])
            for j in range(G):
                pltpu.async_copy(proj_hbm.at[idx_v.at[p].at[j]],
                                 rows_v.at[p].at[pl.ds(j * GW, GW)], sems[p])

        def drain(p):
            for j in range(G):
                pltpu.make_async_copy(
                    proj_hbm.at[idx_v.at[p].at[j]],
                    rows_v.at[p].at[pl.ds(j * GW, GW)], sems[p]).wait()

        def compute(c, p):
            base = pl.multiple_of(wid * ROWS_PER_W + c * CH, CH)
            rbuf = rows_v.at[p]
            for r in range(CH):
                rb = r * SEQ

                def tok_body(t, accs):
                    a0, a1, a2, a3 = accs
                    q = rb + t * 4
                    return (a0 + rbuf[q, :], a1 + rbuf[q + 1, :],
                            a2 + rbuf[q + 2, :], a3 + rbuf[q + 3, :])

                z = jnp.zeros((PAD,), jnp.float32)
                a0, a1, a2, a3 = lax.fori_loop(0, SEQ // 4, tok_body,
                                               (z, z, z, z))
                out_v[r, :] = (a0 + a1) + (a2 + a3)
            pltpu.sync_copy(out_v, out_hbm.at[pl.ds(base, CH)])

        stage(0, 0)

        def pair_body(g, carry):
            stage(2 * g + 1, 1)
            drain(0)
            compute(2 * g, 0)

            @pl.when(g < NCHUNK // 2 - 1)
            def _():
                stage(2 * g + 2, 0)

            drain(1)
            compute(2 * g + 1, 1)
            return carry

        lax.fori_loop(0, NCHUNK // 2, pair_body, 0)

    return k(proj, data_r)


def _fin_body(s_ref, l_ref, b_ref, o_ref):
    o_ref[...] = s_ref[...] / l_ref[...].astype(jnp.float32) + b_ref[...]


def _finalize(sums, length2d, bias16):
    return pl.pallas_call(
        _fin_body,
        out_shape=jax.ShapeDtypeStruct((BATCH, PAD), jnp.float32),
    )(sums, length2d, bias16)


def kernel(data, length, table, W, b):
    data = data.astype(jnp.int32)
    wt = jnp.zeros((EMB, PAD), jnp.float32).at[:, :2].set(W.T)
    m = (jnp.eye(PACK, dtype=jnp.float32)[:, None, :, None]
         * wt[None, :, None, :]).reshape(KP, NP)
    bias16 = jnp.zeros((1, PAD), jnp.float32).at[0, :2].set(b)
    packed = _project(table.reshape(VP, KP), m)
    proj = packed.reshape(VOCAB, PAD)
    data_r = data.reshape(BATCH * SEQ // GW, GW)
    sums = _sc_pool(proj, data_r)
    out = _finalize(sums, length.reshape(BATCH, 1), bias16)
    return out[:, :2]


# speedup vs baseline: 8.5178x; 2.5027x over previous
import functools

import jax
import jax.numpy as jnp
from jax import lax
from jax.experimental import pallas as pl
from jax.experimental.pallas import tpu as pltpu
from jax.experimental.pallas import tpu_sc as plsc

VOCAB = 1000000
EMB = 64
PAD = 16
BATCH = 16384
SEQ = 200

PACK = 8
VP = VOCAB // PACK
NP = PAD * PACK

NC = 2
NS = 16
NW = NC * NS
ROWS_PER_W = BATCH // NW
CH = 8
NCHUNK = ROWS_PER_W // CH
TOK = CH * SEQ
GA, GB = 104, 96
G = 2 * CH

BV = 32768


def _proj_body(t_ref, w_ref, sel_ref, o_ref):
    out8 = lax.dot_general(t_ref[...], w_ref[...],
                           dimension_numbers=(((0,), (0,)), ((), ())),
                           preferred_element_type=jnp.float32)
    out83 = out8.reshape(BV // PACK, PACK, NP)
    o_ref[...] = jnp.sum(out83 * sel_ref[...][None, :, :], axis=1)


def _project(table_t, wt_rep, sel):
    return pl.pallas_call(
        _proj_body,
        grid=((VOCAB + BV - 1) // BV,),
        in_specs=[
            pl.BlockSpec((EMB, BV), lambda i: (0, i)),
            pl.BlockSpec((EMB, NP), lambda i: (0, 0)),
            pl.BlockSpec((PACK, NP), lambda i: (0, 0)),
        ],
        out_specs=pl.BlockSpec((BV // PACK, NP), lambda i: (i, 0)),
        out_shape=jax.ShapeDtypeStruct((VP, NP), jnp.float32),
    )(table_t, wt_rep, sel)


def _sc_pool(proj, data):
    mesh = plsc.VectorSubcoreMesh(core_axis_name="c", subcore_axis_name="s")

    @functools.partial(
        pl.kernel,
        mesh=mesh,
        compiler_params=pltpu.CompilerParams(use_tc_tiling_on_sc=False),
        out_type=jax.ShapeDtypeStruct((BATCH, PAD), jnp.float32),
        scratch_types=[
            pltpu.VMEM((2, CH, SEQ), jnp.int32),
            pltpu.VMEM((2, TOK, PAD), jnp.float32),
            pltpu.VMEM((CH, PAD), jnp.float32),
            pltpu.SemaphoreType.DMA,
            pltpu.SemaphoreType.DMA,
        ],
    )
    def k(proj_hbm, data_hbm, out_hbm, idx_v, rows_v, out_v, sem0, sem1):
        cid = lax.axis_index("c")
        sid = lax.axis_index("s")
        wid = sid * NC + cid
        sems = (sem0, sem1)

        def gather_list(p):
            ib = idx_v.at[p]
            pairs = []
            for r in range(CH):
                pairs.append((proj_hbm.at[ib.at[r, pl.ds(0, GA)]],
                              rows_v.at[p].at[pl.ds(r * SEQ, GA)]))
                pairs.append((proj_hbm.at[ib.at[r, pl.ds(GA, GB)]],
                              rows_v.at[p].at[pl.ds(r * SEQ + GA, GB)]))
            return pairs

        def stage(c, p):
            base = pl.multiple_of(wid * ROWS_PER_W + c * CH, CH)
            pltpu.sync_copy(data_hbm.at[pl.ds(base, CH), :], idx_v.at[p])
            for src, dst in gather_list(p):
                pltpu.async_copy(src, dst, sems[p])

        def drain(p):
            for src, dst in gather_list(p):
                pltpu.make_async_copy(src, dst, sems[p]).wait()

        def compute(c, p):
            base = pl.multiple_of(wid * ROWS_PER_W + c * CH, CH)
            rbuf = rows_v.at[p]
            for r in range(CH):
                rb = r * SEQ

                def tok_body(t, accs):
                    a0, a1, a2, a3 = accs
                    q = rb + t * 4
                    return (a0 + rbuf[q, :], a1 + rbuf[q + 1, :],
                            a2 + rbuf[q + 2, :], a3 + rbuf[q + 3, :])

                z = jnp.zeros((PAD,), jnp.float32)
                a0, a1, a2, a3 = lax.fori_loop(0, SEQ // 4, tok_body,
                                               (z, z, z, z))
                out_v[r, :] = (a0 + a1) + (a2 + a3)
            pltpu.sync_copy(out_v, out_hbm.at[pl.ds(base, CH)])

        stage(0, 0)

        def pair_body(g, carry):
            stage(2 * g + 1, 1)
            drain(0)
            compute(2 * g, 0)

            @pl.when(g < NCHUNK // 2 - 1)
            def _():
                stage(2 * g + 2, 0)

            drain(1)
            compute(2 * g + 1, 1)
            return carry

        lax.fori_loop(0, NCHUNK // 2, pair_body, 0)

    return k(proj, data)


def _fin_body(s_ref, l_ref, b_ref, o_ref):
    o_ref[...] = s_ref[...] / l_ref[...].astype(jnp.float32) + b_ref[...]


def _finalize(sums, length2d, bias16):
    return pl.pallas_call(
        _fin_body,
        out_shape=jax.ShapeDtypeStruct((BATCH, PAD), jnp.float32),
    )(sums, length2d, bias16)


def kernel(data, length, table, W, b):
    data = data.astype(jnp.int32)
    wt = jnp.zeros((EMB, PAD), jnp.float32).at[:, :2].set(W.T)
    wt_rep = jnp.tile(wt, (1, PACK))
    sel = (jnp.arange(NP) // PAD == jnp.arange(PACK)[:, None]
           ).astype(jnp.float32)
    bias16 = jnp.zeros((1, PAD), jnp.float32).at[0, :2].set(b)
    packed = _project(table.T, wt_rep, sel)
    proj = packed.reshape(VOCAB, PAD)
    sums = _sc_pool(proj, data)
    out = _finalize(sums, length.reshape(BATCH, 1), bias16)
    return out[:, :2]
